# baseline (speedup 1.0000x reference)
import jax
import jax.numpy as jnp
from jax import lax
from jax.experimental import pallas as pl
from jax.experimental.pallas import tpu as pltpu
from jax.experimental.pallas import tpu_sc as plsc

NC = 2
NS = 16
L = 16
NW = NC * NS

BATCH = 16384
DIM = 32
BPW = BATCH // NW
GROUPS = BPW // L


def _rsqrt_clip(x):
    i = plsc.bitcast(x, jnp.int32)
    y = plsc.bitcast(jnp.int32(0x5F3759DF) - (i >> 1), jnp.float32)
    for _ in range(3):
        y = y * (1.5 - 0.5 * x * y * y)
    return jnp.minimum(jnp.float32(1.0), y)


def _body(users_hbm, pos_hbm, neg_hbm, ut_hbm, it_hbm, out_hbm,
          idx_u, idx_p, idx_n, du, dp_, dn_, out_v, sem):
    wid = lax.axis_index("s") * NC + lax.axis_index("c")
    base = wid * BPW

    pltpu.sync_copy(users_hbm.at[pl.ds(base, BPW)], idx_u)
    pltpu.sync_copy(pos_hbm.at[pl.ds(base, BPW)], idx_p)
    pltpu.sync_copy(neg_hbm.at[pl.ds(base, BPW)], idx_n)

    for d in range(DIM):
        pltpu.async_copy(ut_hbm.at[d].at[idx_u], du.at[d], sem)
        pltpu.async_copy(it_hbm.at[d].at[idx_p], dp_.at[d], sem)
        pltpu.async_copy(it_hbm.at[d].at[idx_n], dn_.at[d], sem)
    for d in range(DIM):
        pltpu.make_async_copy(ut_hbm.at[d].at[idx_u], du.at[d], sem).wait()
        pltpu.make_async_copy(it_hbm.at[d].at[idx_p], dp_.at[d], sem).wait()
        pltpu.make_async_copy(it_hbm.at[d].at[idx_n], dn_.at[d], sem).wait()

    @pl.loop(0, GROUPS)
    def _group(g):
        j0 = g * L
        zero = jnp.zeros((L,), jnp.float32)
        nu = zero
        npp = zero
        nn = zero
        dpos = zero
        dneg = zero
        for d in range(DIM):
            u = du[d, pl.ds(j0, L)]
            p = dp_[d, pl.ds(j0, L)]
            n = dn_[d, pl.ds(j0, L)]
            nu = nu + u * u
            npp = npp + p * p
            nn = nn + n * n
            dpos = dpos + u * p
            dneg = dneg + u * n
        su = _rsqrt_clip(nu)
        sp = _rsqrt_clip(npp)
        sn = _rsqrt_clip(nn)
        out_v[pl.ds(j0, L)] = su * (dpos * sp - dneg * sn)

    pltpu.sync_copy(out_v, out_hbm.at[pl.ds(base, BPW)])


def kernel(users, pos_items, neg_items, user_table, item_table):
    mesh = plsc.VectorSubcoreMesh(
        core_axis_name="c", subcore_axis_name="s",
        num_cores=NC, num_subcores=NS)
    f = pl.kernel(
        _body,
        out_type=jax.ShapeDtypeStruct((BATCH,), jnp.float32),
        mesh=mesh,
        compiler_params=pltpu.CompilerParams(
            needs_layout_passes=False, use_tc_tiling_on_sc=False),
        scratch_types=[
            pltpu.VMEM((BPW,), jnp.int32),
            pltpu.VMEM((BPW,), jnp.int32),
            pltpu.VMEM((BPW,), jnp.int32),
            pltpu.VMEM((DIM, BPW), jnp.float32),
            pltpu.VMEM((DIM, BPW), jnp.float32),
            pltpu.VMEM((DIM, BPW), jnp.float32),
            pltpu.VMEM((BPW,), jnp.float32),
            pltpu.SemaphoreType.DMA,
        ],
    )
    return f(users, pos_items, neg_items, user_table.T, item_table.T)


# speedup vs baseline: 1.3209x; 1.3209x over previous
import jax
import jax.numpy as jnp
from jax import lax
from jax.experimental import pallas as pl
from jax.experimental.pallas import tpu as pltpu
from jax.experimental.pallas import tpu_sc as plsc

NC = 2
NS = 16
L = 16
NW = NC * NS

BATCH = 16384
DIM = 32
NROWS = 1000000
BPW = BATCH // NW
GROUPS = BPW // L

TPAD = 7816
ROWLEN = TPAD * 128


def _detile_body(x_ref, y_ref):
    y_ref[...] = x_ref[...].reshape(8, 8, 128)


def _detile(table_t):
    return pl.pallas_call(
        _detile_body,
        grid=(DIM // 8, TPAD // 8),
        in_specs=[pl.BlockSpec((8, 1024), lambda i, j: (i, j))],
        out_specs=pl.BlockSpec((8, 8, 128), lambda i, j: (i, j, 0)),
        out_shape=jax.ShapeDtypeStruct((DIM, TPAD, 128), jnp.float32),
    )(table_t)


def _rsqrt_clip(x):
    i = plsc.bitcast(x, jnp.int32)
    y = plsc.bitcast(jnp.int32(0x5F3759DF) - (i >> 1), jnp.float32)
    for _ in range(3):
        y = y * (1.5 - 0.5 * x * y * y)
    return jnp.minimum(jnp.float32(1.0), y)


def _body(users_hbm, pos_hbm, neg_hbm, ut_hbm, it_hbm, out_hbm,
          idx_u, idx_p, idx_n, cu, cp, cn, out_v,
          sem_u, sem_p, sem_n):
    wid = lax.axis_index("s") * NC + lax.axis_index("c")
    base = wid * BPW

    pltpu.sync_copy(users_hbm.at[pl.ds(base, BPW)], idx_u)
    pltpu.sync_copy(pos_hbm.at[pl.ds(base, BPW)], idx_p)
    pltpu.sync_copy(neg_hbm.at[pl.ds(base, BPW)], idx_n)

    copies = []
    for d in range(DIM):
        row_u = ut_hbm.at[pl.ds(d * ROWLEN, ROWLEN)]
        row_i = it_hbm.at[pl.ds(d * ROWLEN, ROWLEN)]
        copies.append(pltpu.async_copy(
            row_u.at[idx_u], cu.at[pl.ds(d * BPW, BPW)], sem_u))
        copies.append(pltpu.async_copy(
            row_i.at[idx_p], cp.at[pl.ds(d * BPW, BPW)], sem_p))
        copies.append(pltpu.async_copy(
            row_i.at[idx_n], cn.at[pl.ds(d * BPW, BPW)], sem_n))
    for c in copies:
        c.wait()

    @pl.loop(0, GROUPS)
    def _group(g):
        zero = jnp.zeros((L,), jnp.float32)
        nu = zero
        npp = zero
        nn = zero
        dp = zero
        dn = zero
        for d in range(DIM):
            u = cu[pl.ds(d * BPW + g * L, L)]
            p = cp[pl.ds(d * BPW + g * L, L)]
            n = cn[pl.ds(d * BPW + g * L, L)]
            nu = nu + u * u
            npp = npp + p * p
            nn = nn + n * n
            dp = dp + u * p
            dn = dn + u * n
        su = _rsqrt_clip(nu)
        sp = _rsqrt_clip(npp)
        sn = _rsqrt_clip(nn)
        out_v[pl.ds(g * L, L)] = su * (dp * sp - dn * sn)

    pltpu.sync_copy(out_v, out_hbm.at[pl.ds(base, BPW)])


def kernel(users, pos_items, neg_items, user_table, item_table):
    ut_lin = _detile(user_table.T).reshape(DIM * ROWLEN)
    it_lin = _detile(item_table.T).reshape(DIM * ROWLEN)

    mesh = plsc.VectorSubcoreMesh(
        core_axis_name="c", subcore_axis_name="s",
        num_cores=NC, num_subcores=NS)
    f = pl.kernel(
        _body,
        out_type=jax.ShapeDtypeStruct((BATCH,), jnp.float32),
        mesh=mesh,
        compiler_params=pltpu.CompilerParams(
            needs_layout_passes=False, use_tc_tiling_on_sc=False),
        scratch_types=[
            pltpu.VMEM((BPW,), jnp.int32),
            pltpu.VMEM((BPW,), jnp.int32),
            pltpu.VMEM((BPW,), jnp.int32),
            pltpu.VMEM((DIM * BPW,), jnp.float32),
            pltpu.VMEM((DIM * BPW,), jnp.float32),
            pltpu.VMEM((DIM * BPW,), jnp.float32),
            pltpu.VMEM((BPW,), jnp.float32),
            pltpu.SemaphoreType.DMA,
            pltpu.SemaphoreType.DMA,
            pltpu.SemaphoreType.DMA,
        ],
    )
    return f(users, pos_items, neg_items, ut_lin, it_lin)


# speedup vs baseline: 17.2590x; 13.0658x over previous
import jax
import jax.numpy as jnp
import numpy as np
from jax import lax
from jax.experimental import pallas as pl
from jax.experimental.pallas import tpu as pltpu
from jax.experimental.pallas import tpu_sc as plsc

NC = 2
NS = 16
L = 16
NW = NC * NS

BATCH = 16384
DIM = 32
PDIM = DIM // 2
NROWS = 1000000
BPW = BATCH // NW
GROUPS = BPW // L

TPAD = 7936
ROWLEN = TPAD * 128
TW = 63488

MASKH = np.int32(-65536)
MASKL = np.int32(0xFFFF)


def _pack_body(x_ref, y_ref):
    x = lax.bitcast_convert_type(x_ref[...], jnp.int32)
    x2 = x.reshape(PDIM, 2, TW)
    a = x2[:, 0, :]
    b = x2[:, 1, :]
    rn = jnp.int32(0x7FFF)
    a = a + rn + ((a >> 16) & 1)
    b = b + rn + ((b >> 16) & 1)
    packed = (a & MASKH) | ((b >> 16) & MASKL)
    y_ref[...] = packed.reshape(PDIM, TW // 128, 128)


def _pack(table_t):
    return pl.pallas_call(
        _pack_body,
        grid=(ROWLEN // TW,),
        in_specs=[pl.BlockSpec((DIM, TW), lambda j: (0, j))],
        out_specs=pl.BlockSpec((PDIM, TW // 128, 128), lambda j: (0, j, 0)),
        out_shape=jax.ShapeDtypeStruct((PDIM, TPAD, 128), jnp.int32),
        compiler_params=pltpu.CompilerParams(
            dimension_semantics=("parallel",)),
    )(table_t)


def _rsqrt_clip(x):
    i = plsc.bitcast(x, jnp.int32)
    y = plsc.bitcast(jnp.int32(0x5F3759DF) - (i >> 1), jnp.float32)
    for _ in range(3):
        y = y * (1.5 - 0.5 * x * y * y)
    return jnp.minimum(jnp.float32(1.0), y)


def _unpack(v):
    fa = plsc.bitcast(v & MASKH, jnp.float32)
    fb = plsc.bitcast(v << 16, jnp.float32)
    return fa, fb


def _body(users_hbm, pos_hbm, neg_hbm, ut_hbm, it_hbm, out_hbm,
          idx_u, idx_p, idx_n, cu, cp, cn, out_v,
          sem_u, sem_p, sem_n):
    wid = lax.axis_index("s") * NC + lax.axis_index("c")
    base = wid * BPW

    pltpu.sync_copy(users_hbm.at[pl.ds(base, BPW)], idx_u)
    pltpu.sync_copy(pos_hbm.at[pl.ds(base, BPW)], idx_p)
    pltpu.sync_copy(neg_hbm.at[pl.ds(base, BPW)], idx_n)

    copies = []
    for k in range(PDIM):
        row_u = ut_hbm.at[pl.ds(k * ROWLEN, ROWLEN)]
        row_i = it_hbm.at[pl.ds(k * ROWLEN, ROWLEN)]
        copies.append(pltpu.async_copy(
            row_u.at[idx_u], cu.at[pl.ds(k * BPW, BPW)], sem_u))
        copies.append(pltpu.async_copy(
            row_i.at[idx_p], cp.at[pl.ds(k * BPW, BPW)], sem_p))
        copies.append(pltpu.async_copy(
            row_i.at[idx_n], cn.at[pl.ds(k * BPW, BPW)], sem_n))
    for c in copies:
        c.wait()

    @pl.loop(0, GROUPS)
    def _group(g):
        zero = jnp.zeros((L,), jnp.float32)
        nu = zero
        npp = zero
        nn = zero
        dp = zero
        dn = zero
        for k in range(PDIM):
            ua, ub = _unpack(cu[pl.ds(k * BPW + g * L, L)])
            pa, pb = _unpack(cp[pl.ds(k * BPW + g * L, L)])
            na, nb = _unpack(cn[pl.ds(k * BPW + g * L, L)])
            nu = nu + ua * ua + ub * ub
            npp = npp + pa * pa + pb * pb
            nn = nn + na * na + nb * nb
            dp = dp + ua * pa + ub * pb
            dn = dn + ua * na + ub * nb
        su = _rsqrt_clip(nu)
        sp = _rsqrt_clip(npp)
        sn = _rsqrt_clip(nn)
        out_v[pl.ds(g * L, L)] = su * (dp * sp - dn * sn)

    pltpu.sync_copy(out_v, out_hbm.at[pl.ds(base, BPW)])


def kernel(users, pos_items, neg_items, user_table, item_table):
    ut_pk = _pack(user_table.T).reshape(PDIM * ROWLEN)
    it_pk = _pack(item_table.T).reshape(PDIM * ROWLEN)

    mesh = plsc.VectorSubcoreMesh(
        core_axis_name="c", subcore_axis_name="s",
        num_cores=NC, num_subcores=NS)
    f = pl.kernel(
        _body,
        out_type=jax.ShapeDtypeStruct((BATCH,), jnp.float32),
        mesh=mesh,
        compiler_params=pltpu.CompilerParams(
            needs_layout_passes=False, use_tc_tiling_on_sc=False),
        scratch_types=[
            pltpu.VMEM((BPW,), jnp.int32),
            pltpu.VMEM((BPW,), jnp.int32),
            pltpu.VMEM((BPW,), jnp.int32),
            pltpu.VMEM((PDIM * BPW,), jnp.int32),
            pltpu.VMEM((PDIM * BPW,), jnp.int32),
            pltpu.VMEM((PDIM * BPW,), jnp.int32),
            pltpu.VMEM((BPW,), jnp.float32),
            pltpu.SemaphoreType.DMA,
            pltpu.SemaphoreType.DMA,
            pltpu.SemaphoreType.DMA,
        ],
    )
    return f(users, pos_items, neg_items, ut_pk, it_pk)
